# initial kernel scaffold (unmeasured)
import jax
import jax.numpy as jnp
from jax import lax
from jax.experimental import pallas as pl
from jax.experimental.pallas import tpu as pltpu

N_DEV = 4
SCALE = 0.08838834764831843
BLK = 64


def kernel(x, Wq, K_ext, V_ext, Wo):
    B, Sq, D = x.shape
    _, Skv, Hl, Dh = K_ext.shape
    Dm = Hl * Dh
    Dout = Wo.shape[1]

    def body(x_ref, wq_ref, k_ref, v_ref, wo_ref, out_ref,
             wq_s, wo_s, comm_ref, send_sems, recv_sems):
        my = lax.axis_index("i")

        barrier = pltpu.get_barrier_semaphore()
        for k in range(1, N_DEV):
            pl.semaphore_signal(
                barrier, inc=1,
                device_id=(lax.rem(my + k, N_DEV),),
                device_id_type=pl.DeviceIdType.MESH,
            )
        pl.semaphore_wait(barrier, N_DEV - 1)

        for g in range(N_DEV):
            @pl.when(my == g)
            def _(g=g):
                wq_s[...] = wq_ref[:, g * Dm:(g + 1) * Dm].astype(jnp.bfloat16)
                wo_s[...] = wo_ref[g * Dm:(g + 1) * Dm, :].astype(jnp.bfloat16)

        xb = x_ref[0].astype(jnp.bfloat16)
        q = jnp.dot(xb, wq_s[...], preferred_element_type=jnp.float32)
        q = (q * SCALE).astype(jnp.bfloat16)

        rb = lax.broadcasted_iota(jnp.int32, (Sq, Skv), 0) // BLK
        cb = lax.broadcasted_iota(jnp.int32, (Sq, Skv), 1) // BLK
        mask = (cb % 4) == rb

        acc = jnp.zeros((Sq, Dout), jnp.float32)
        for h in range(Hl):
            qh = q[:, h * Dh:(h + 1) * Dh]
            kh = k_ref[0, :, h, :].astype(jnp.bfloat16)
            s = lax.dot_general(qh, kh, (((1,), (1,)), ((), ())),
                                preferred_element_type=jnp.float32)
            s = jnp.where(mask, s, -1e9)
            m = jnp.max(s, axis=1, keepdims=True)
            w = jnp.exp(s - m)
            w = (w / jnp.sum(w, axis=1, keepdims=True)).astype(jnp.bfloat16)
            vh = v_ref[0, :, h, :].astype(jnp.bfloat16)
            ctx = jnp.dot(w, vh, preferred_element_type=jnp.float32)
            acc = acc + jnp.dot(ctx.astype(jnp.bfloat16),
                                wo_s[h * Dh:(h + 1) * Dh, :],
                                preferred_element_type=jnp.float32)

        comm_ref[0] = acc.astype(jnp.bfloat16)

        sends = []
        for k in range(1, N_DEV):
            rdma = pltpu.make_async_remote_copy(
                src_ref=comm_ref.at[0],
                dst_ref=comm_ref.at[k],
                send_sem=send_sems.at[k - 1],
                recv_sem=recv_sems.at[k],
                device_id=(lax.rem(my + k, N_DEV),),
                device_id_type=pl.DeviceIdType.MESH,
            )
            rdma.start()
            sends.append(rdma)

        for k in range(1, N_DEV):
            recv = pltpu.make_async_remote_copy(
                src_ref=comm_ref.at[0],
                dst_ref=comm_ref.at[k],
                send_sem=send_sems.at[k - 1],
                recv_sem=recv_sems.at[k],
                device_id=(lax.rem(my + k, N_DEV),),
                device_id_type=pl.DeviceIdType.MESH,
            )
            recv.wait_recv()
        for rdma in sends:
            rdma.wait_send()

        total = (comm_ref[0].astype(jnp.float32)
                 + comm_ref[1].astype(jnp.float32)
                 + comm_ref[2].astype(jnp.float32)
                 + comm_ref[3].astype(jnp.float32))
        out_ref[0] = total

    return pl.pallas_call(
        body,
        out_shape=jax.ShapeDtypeStruct((B, Sq, Dout), jnp.float32),
        in_specs=[pl.BlockSpec(memory_space=pltpu.VMEM)] * 5,
        out_specs=pl.BlockSpec(memory_space=pltpu.VMEM),
        scratch_shapes=[
            pltpu.VMEM((D, Dm), jnp.bfloat16),
            pltpu.VMEM((Dm, Dout), jnp.bfloat16),
            pltpu.VMEM((N_DEV, Sq, Dout), jnp.bfloat16),
            pltpu.SemaphoreType.DMA((N_DEV - 1,)),
            pltpu.SemaphoreType.DMA((N_DEV,)),
        ],
        compiler_params=pltpu.CompilerParams(collective_id=0),
    )(x, Wq, K_ext, V_ext, Wo)


# baseline (device time: 65708 ns/iter reference)
import jax
import jax.numpy as jnp
from jax import lax
from jax.experimental import pallas as pl
from jax.experimental.pallas import tpu as pltpu

N_DEV = 4
SCALE = 0.08838834764831843
BLK = 64


def kernel(x, Wq, K_ext, V_ext, Wo):
    B, Sq, D = x.shape
    _, Skv, Hl, Dh = K_ext.shape
    Dm = Hl * Dh
    Dout = Wo.shape[1]

    Wq3 = Wq.reshape(D, N_DEV, Dm)
    Wo3 = Wo.reshape(N_DEV, Dm, Dout)

    def body(x_ref, wq_ref, k_ref, v_ref, wo_ref, out_ref,
             wq_s, wo_s, k_s, v_s, comm_ref, copy_sems, send_sems, recv_sems):
        my = lax.axis_index("i")

        barrier = pltpu.get_barrier_semaphore()
        for k in range(1, N_DEV):
            pl.semaphore_signal(
                barrier, inc=1,
                device_id=(lax.rem(my + k, N_DEV),),
                device_id_type=pl.DeviceIdType.MESH,
            )
        pl.semaphore_wait(barrier, N_DEV - 1)

        wq_cp = pltpu.make_async_copy(wq_ref.at[:, my, :], wq_s, copy_sems.at[0])
        wq_cp.start()
        wo_cp = pltpu.make_async_copy(wo_ref.at[my], wo_s, copy_sems.at[1])
        wo_cp.start()

        def kv_copies(h, slot):
            kc = pltpu.make_async_copy(
                k_ref.at[0, :, h, :], k_s.at[slot], copy_sems.at[2 + slot])
            vc = pltpu.make_async_copy(
                v_ref.at[0, :, h, :], v_s.at[slot], copy_sems.at[4 + slot])
            return kc, vc

        kc0, vc0 = kv_copies(0, 0)
        kc0.start()
        vc0.start()

        wq_cp.wait()
        xb = x_ref[0].astype(jnp.bfloat16)
        q = jnp.dot(xb, wq_s[...].astype(jnp.bfloat16),
                    preferred_element_type=jnp.float32)
        q = (q * SCALE).astype(jnp.bfloat16)

        rb = lax.broadcasted_iota(jnp.int32, (Sq, Skv), 0) // BLK
        cb = lax.broadcasted_iota(jnp.int32, (Sq, Skv), 1) // BLK
        mask = (cb % 4) == rb

        wo_cp.wait()
        wo_b = wo_s[...].astype(jnp.bfloat16)

        acc = jnp.zeros((Sq, Dout), jnp.float32)
        for h in range(Hl):
            cur = h % 2
            if h + 1 < Hl:
                kcn, vcn = kv_copies(h + 1, (h + 1) % 2)
                kcn.start()
                vcn.start()
            kc, vc = kv_copies(h, cur)
            kc.wait()
            vc.wait()

            qh = q[:, h * Dh:(h + 1) * Dh]
            kh = k_s[cur].astype(jnp.bfloat16)
            s = lax.dot_general(qh, kh, (((1,), (1,)), ((), ())),
                                preferred_element_type=jnp.float32)
            s = jnp.where(mask, s, -1e9)
            m = jnp.max(s, axis=1, keepdims=True)
            w = jnp.exp(s - m)
            w = (w / jnp.sum(w, axis=1, keepdims=True)).astype(jnp.bfloat16)
            vh = v_s[cur].astype(jnp.bfloat16)
            ctx = jnp.dot(w, vh, preferred_element_type=jnp.float32)
            acc = acc + jnp.dot(ctx.astype(jnp.bfloat16),
                                wo_b[h * Dh:(h + 1) * Dh, :],
                                preferred_element_type=jnp.float32)

        comm_ref[0] = acc.astype(jnp.bfloat16)

        sends = []
        for k in range(1, N_DEV):
            rdma = pltpu.make_async_remote_copy(
                src_ref=comm_ref.at[0],
                dst_ref=comm_ref.at[k],
                send_sem=send_sems.at[k - 1],
                recv_sem=recv_sems.at[k],
                device_id=(lax.rem(my + k, N_DEV),),
                device_id_type=pl.DeviceIdType.MESH,
            )
            rdma.start()
            sends.append(rdma)

        for k in range(1, N_DEV):
            recv = pltpu.make_async_remote_copy(
                src_ref=comm_ref.at[0],
                dst_ref=comm_ref.at[k],
                send_sem=send_sems.at[k - 1],
                recv_sem=recv_sems.at[k],
                device_id=(lax.rem(my + k, N_DEV),),
                device_id_type=pl.DeviceIdType.MESH,
            )
            recv.wait_recv()
        for rdma in sends:
            rdma.wait_send()

        total = (comm_ref[0].astype(jnp.float32)
                 + comm_ref[1].astype(jnp.float32)
                 + comm_ref[2].astype(jnp.float32)
                 + comm_ref[3].astype(jnp.float32))
        out_ref[0] = total

    return pl.pallas_call(
        body,
        out_shape=jax.ShapeDtypeStruct((B, Sq, Dout), jnp.float32),
        in_specs=[
            pl.BlockSpec(memory_space=pltpu.VMEM),
            pl.BlockSpec(memory_space=pl.ANY),
            pl.BlockSpec(memory_space=pl.ANY),
            pl.BlockSpec(memory_space=pl.ANY),
            pl.BlockSpec(memory_space=pl.ANY),
        ],
        out_specs=pl.BlockSpec(memory_space=pltpu.VMEM),
        scratch_shapes=[
            pltpu.VMEM((D, Dm), jnp.float32),
            pltpu.VMEM((Dm, Dout), jnp.float32),
            pltpu.VMEM((2, Skv, Dh), jnp.float32),
            pltpu.VMEM((2, Skv, Dh), jnp.float32),
            pltpu.VMEM((N_DEV, Sq, Dout), jnp.bfloat16),
            pltpu.SemaphoreType.DMA((6,)),
            pltpu.SemaphoreType.DMA((N_DEV - 1,)),
            pltpu.SemaphoreType.DMA((N_DEV,)),
        ],
        compiler_params=pltpu.CompilerParams(
            collective_id=0, vmem_limit_bytes=56 * 1024 * 1024,
        ),
    )(x, Wq3, K_ext, V_ext, Wo3)


# device time: 60992 ns/iter; 1.0773x vs baseline; 1.0773x over previous
import jax
import jax.numpy as jnp
from jax import lax
from jax.experimental import pallas as pl
from jax.experimental.pallas import tpu as pltpu

N_DEV = 4
SCALE = 0.08838834764831843
BLK = 64
NQB = 4


def kernel(x, Wq, K_ext, V_ext, Wo):
    B, Sq, D = x.shape
    _, Skv, Hl, Dh = K_ext.shape
    Dm = Hl * Dh
    Dout = Wo.shape[1]
    NC = Skv // (NQB * BLK)

    Wq3 = Wq.reshape(D, N_DEV, Dm)
    Wo3 = Wo.reshape(N_DEV, Dm, Dout)
    K5 = K_ext.reshape(NC, NQB, BLK, Hl, Dh)
    V5 = V_ext.reshape(NC, NQB, BLK, Hl, Dh)

    def body(x_ref, wq_ref, k_ref, v_ref, wo_ref, out_ref,
             wq_s, wo_s, k_s, v_s, comm_ref, w_sems, kv_sems,
             send_sems, recv_sems):
        my = lax.axis_index("i")

        barrier = pltpu.get_barrier_semaphore()
        for k in range(1, N_DEV):
            pl.semaphore_signal(
                barrier, inc=1,
                device_id=(lax.rem(my + k, N_DEV),),
                device_id_type=pl.DeviceIdType.MESH,
            )
        pl.semaphore_wait(barrier, N_DEV - 1)

        wq_cp = pltpu.make_async_copy(wq_ref.at[:, my, :], wq_s, w_sems.at[0])
        wq_cp.start()
        wo_cp = pltpu.make_async_copy(wo_ref.at[my], wo_s, w_sems.at[1])
        wo_cp.start()

        def panel_copies(qb, slot):
            cps = []
            for h in range(Hl):
                cps.append(pltpu.make_async_copy(
                    k_ref.at[:, qb, :, h, :], k_s.at[slot, h],
                    kv_sems.at[slot, h]))
                cps.append(pltpu.make_async_copy(
                    v_ref.at[:, qb, :, h, :], v_s.at[slot, h],
                    kv_sems.at[slot, Hl + h]))
            return cps

        for cp in panel_copies(0, 0):
            cp.start()

        wq_cp.wait()
        xb = x_ref[0].astype(jnp.bfloat16)
        q = jnp.dot(xb, wq_s[...].astype(jnp.bfloat16),
                    preferred_element_type=jnp.float32)
        q = (q * SCALE).astype(jnp.bfloat16)
        wo_cp.wait()
        wo_b = wo_s[...].astype(jnp.bfloat16)

        sends = []
        for qb in range(NQB):
            slot = qb % 2
            if qb + 1 < NQB:
                for cp in panel_copies(qb + 1, (qb + 1) % 2):
                    cp.start()
            for cp in panel_copies(qb, slot):
                cp.wait()

            acc = jnp.zeros((BLK, Dout), jnp.float32)
            for h in range(Hl):
                qh = q[qb * BLK:(qb + 1) * BLK, h * Dh:(h + 1) * Dh]
                kh = k_s[slot, h].reshape(NC * BLK, Dh).astype(jnp.bfloat16)
                s = lax.dot_general(qh, kh, (((1,), (1,)), ((), ())),
                                    preferred_element_type=jnp.float32)
                m = jnp.max(s, axis=1, keepdims=True)
                w = jnp.exp(s - m)
                w = (w / jnp.sum(w, axis=1, keepdims=True)).astype(jnp.bfloat16)
                vh = v_s[slot, h].reshape(NC * BLK, Dh).astype(jnp.bfloat16)
                ctx = jnp.dot(w, vh, preferred_element_type=jnp.float32)
                acc = acc + jnp.dot(ctx.astype(jnp.bfloat16),
                                    wo_b[h * Dh:(h + 1) * Dh, :],
                                    preferred_element_type=jnp.float32)

            comm_ref[0, qb * BLK:(qb + 1) * BLK, :] = acc.astype(jnp.bfloat16)
            for k in range(1, N_DEV):
                rdma = pltpu.make_async_remote_copy(
                    src_ref=comm_ref.at[0, qb * BLK:(qb + 1) * BLK, :],
                    dst_ref=comm_ref.at[k, qb * BLK:(qb + 1) * BLK, :],
                    send_sem=send_sems.at[(k - 1) * NQB + qb],
                    recv_sem=recv_sems.at[k, qb],
                    device_id=(lax.rem(my + k, N_DEV),),
                    device_id_type=pl.DeviceIdType.MESH,
                )
                rdma.start()
                sends.append(rdma)

        for k in range(1, N_DEV):
            for qb in range(NQB):
                recv = pltpu.make_async_remote_copy(
                    src_ref=comm_ref.at[0, qb * BLK:(qb + 1) * BLK, :],
                    dst_ref=comm_ref.at[k, qb * BLK:(qb + 1) * BLK, :],
                    send_sem=send_sems.at[(k - 1) * NQB + qb],
                    recv_sem=recv_sems.at[k, qb],
                    device_id=(lax.rem(my + k, N_DEV),),
                    device_id_type=pl.DeviceIdType.MESH,
                )
                recv.wait_recv()
        for rdma in sends:
            rdma.wait_send()

        total = (comm_ref[0].astype(jnp.float32)
                 + comm_ref[1].astype(jnp.float32)
                 + comm_ref[2].astype(jnp.float32)
                 + comm_ref[3].astype(jnp.float32))
        out_ref[0] = total

    return pl.pallas_call(
        body,
        out_shape=jax.ShapeDtypeStruct((B, Sq, Dout), jnp.float32),
        in_specs=[
            pl.BlockSpec(memory_space=pltpu.VMEM),
            pl.BlockSpec(memory_space=pl.ANY),
            pl.BlockSpec(memory_space=pl.ANY),
            pl.BlockSpec(memory_space=pl.ANY),
            pl.BlockSpec(memory_space=pl.ANY),
        ],
        out_specs=pl.BlockSpec(memory_space=pltpu.VMEM),
        scratch_shapes=[
            pltpu.VMEM((D, Dm), jnp.float32),
            pltpu.VMEM((Dm, Dout), jnp.float32),
            pltpu.VMEM((2, Hl, NC, BLK, Dh), jnp.float32),
            pltpu.VMEM((2, Hl, NC, BLK, Dh), jnp.float32),
            pltpu.VMEM((N_DEV, Sq, Dout), jnp.bfloat16),
            pltpu.SemaphoreType.DMA((2,)),
            pltpu.SemaphoreType.DMA((2, 2 * 8)),
            pltpu.SemaphoreType.DMA(((N_DEV - 1) * NQB,)),
            pltpu.SemaphoreType.DMA((N_DEV, NQB)),
        ],
        compiler_params=pltpu.CompilerParams(
            collective_id=0, vmem_limit_bytes=56 * 1024 * 1024,
        ),
    )(x, Wq3, K5, V5, Wo3)


# device time: 39280 ns/iter; 1.6728x vs baseline; 1.5527x over previous
import jax
import jax.numpy as jnp
from jax import lax
from jax.experimental import pallas as pl
from jax.experimental.pallas import tpu as pltpu

N_DEV = 4
SCALE = 0.08838834764831843
BLK = 64
NQB = 4


def kernel(x, Wq, K_ext, V_ext, Wo):
    B, Sq, D = x.shape
    _, Skv, Hl, Dh = K_ext.shape
    Dm = Hl * Dh
    Dout = Wo.shape[1]
    NC = Skv // (NQB * BLK)

    Wo3 = Wo.reshape(N_DEV, Dm, Dout)
    K5 = K_ext.reshape(NC, NQB, BLK, Hl, Dh)
    V5 = V_ext.reshape(NC, NQB, BLK, Hl, Dh)

    def body(x_ref, wq_ref, k_ref, v_ref, wo_ref, out_ref,
             wq_s, wo_s, k_s, v_s, comm_ref, w_sems, kv_sems,
             send_sems, recv_sems):
        my = lax.axis_index("i")

        barrier = pltpu.get_barrier_semaphore()
        for k in range(1, N_DEV):
            pl.semaphore_signal(
                barrier, inc=1,
                device_id=(lax.rem(my + k, N_DEV),),
                device_id_type=pl.DeviceIdType.MESH,
            )
        pl.semaphore_wait(barrier, N_DEV - 1)

        for g in range(N_DEV):
            @pl.when(my == g)
            def _(g=g):
                pltpu.make_async_copy(
                    wq_ref.at[:, g * Dm:(g + 1) * Dm], wq_s, w_sems.at[0]
                ).start()
        wq_cp = pltpu.make_async_copy(
            wq_ref.at[:, 0:Dm], wq_s, w_sems.at[0])
        wo_cp = pltpu.make_async_copy(wo_ref.at[my], wo_s, w_sems.at[1])
        wo_cp.start()

        def panel_copies(qb, slot):
            cps = []
            for h in range(Hl):
                cps.append(pltpu.make_async_copy(
                    k_ref.at[:, qb, :, h, :], k_s.at[slot, h],
                    kv_sems.at[slot, h]))
                cps.append(pltpu.make_async_copy(
                    v_ref.at[:, qb, :, h, :], v_s.at[slot, h],
                    kv_sems.at[slot, Hl + h]))
            return cps

        for cp in panel_copies(0, 0):
            cp.start()

        wq_cp.wait()
        xb = x_ref[0].astype(jnp.bfloat16)
        q = jnp.dot(xb, wq_s[...].astype(jnp.bfloat16),
                    preferred_element_type=jnp.float32)
        q = (q * SCALE).astype(jnp.bfloat16)
        wo_cp.wait()
        wo_b = wo_s[...].astype(jnp.bfloat16)

        sends = []
        for qb in range(NQB):
            slot = qb % 2
            if qb + 1 < NQB:
                for cp in panel_copies(qb + 1, (qb + 1) % 2):
                    cp.start()
            for cp in panel_copies(qb, slot):
                cp.wait()

            acc = jnp.zeros((BLK, Dout), jnp.float32)
            for h in range(Hl):
                qh = q[qb * BLK:(qb + 1) * BLK, h * Dh:(h + 1) * Dh]
                kh = k_s[slot, h].reshape(NC * BLK, Dh).astype(jnp.bfloat16)
                s = lax.dot_general(qh, kh, (((1,), (1,)), ((), ())),
                                    preferred_element_type=jnp.float32)
                m = jnp.max(s, axis=1, keepdims=True)
                w = jnp.exp(s - m)
                w = (w / jnp.sum(w, axis=1, keepdims=True)).astype(jnp.bfloat16)
                vh = v_s[slot, h].reshape(NC * BLK, Dh).astype(jnp.bfloat16)
                ctx = jnp.dot(w, vh, preferred_element_type=jnp.float32)
                acc = acc + jnp.dot(ctx.astype(jnp.bfloat16),
                                    wo_b[h * Dh:(h + 1) * Dh, :],
                                    preferred_element_type=jnp.float32)

            comm_ref[0, qb * BLK:(qb + 1) * BLK, :] = acc.astype(jnp.bfloat16)
            for k in range(1, N_DEV):
                rdma = pltpu.make_async_remote_copy(
                    src_ref=comm_ref.at[0, qb * BLK:(qb + 1) * BLK, :],
                    dst_ref=comm_ref.at[k, qb * BLK:(qb + 1) * BLK, :],
                    send_sem=send_sems.at[(k - 1) * NQB + qb],
                    recv_sem=recv_sems.at[k, qb],
                    device_id=(lax.rem(my + k, N_DEV),),
                    device_id_type=pl.DeviceIdType.MESH,
                )
                rdma.start()
                sends.append(rdma)

        for k in range(1, N_DEV):
            for qb in range(NQB):
                recv = pltpu.make_async_remote_copy(
                    src_ref=comm_ref.at[0, qb * BLK:(qb + 1) * BLK, :],
                    dst_ref=comm_ref.at[k, qb * BLK:(qb + 1) * BLK, :],
                    send_sem=send_sems.at[(k - 1) * NQB + qb],
                    recv_sem=recv_sems.at[k, qb],
                    device_id=(lax.rem(my + k, N_DEV),),
                    device_id_type=pl.DeviceIdType.MESH,
                )
                recv.wait_recv()
        for rdma in sends:
            rdma.wait_send()

        total = (comm_ref[0].astype(jnp.float32)
                 + comm_ref[1].astype(jnp.float32)
                 + comm_ref[2].astype(jnp.float32)
                 + comm_ref[3].astype(jnp.float32))
        out_ref[0] = total

    return pl.pallas_call(
        body,
        out_shape=jax.ShapeDtypeStruct((B, Sq, Dout), jnp.float32),
        in_specs=[
            pl.BlockSpec(memory_space=pltpu.VMEM),
            pl.BlockSpec(memory_space=pl.ANY),
            pl.BlockSpec(memory_space=pl.ANY),
            pl.BlockSpec(memory_space=pl.ANY),
            pl.BlockSpec(memory_space=pl.ANY),
        ],
        out_specs=pl.BlockSpec(memory_space=pltpu.VMEM),
        scratch_shapes=[
            pltpu.VMEM((D, Dm), jnp.float32),
            pltpu.VMEM((Dm, Dout), jnp.float32),
            pltpu.VMEM((2, Hl, NC, BLK, Dh), jnp.float32),
            pltpu.VMEM((2, Hl, NC, BLK, Dh), jnp.float32),
            pltpu.VMEM((N_DEV, Sq, Dout), jnp.bfloat16),
            pltpu.SemaphoreType.DMA((2,)),
            pltpu.SemaphoreType.DMA((2, 2 * 8)),
            pltpu.SemaphoreType.DMA(((N_DEV - 1) * NQB,)),
            pltpu.SemaphoreType.DMA((N_DEV, NQB)),
        ],
        compiler_params=pltpu.CompilerParams(
            collective_id=0, vmem_limit_bytes=56 * 1024 * 1024,
        ),
    )(x, Wq, K5, V5, Wo3)
